# baseline (speedup 1.0000x reference)
import jax
import jax.numpy as jnp
from jax.experimental import pallas as pl
from jax.experimental.pallas import tpu as pltpu

_N_PER = 32
_BLOCK = 8192
_HALF = _BLOCK // 2


def _first_argmin_t(dist, k):
    m = jnp.min(dist, axis=0, keepdims=True)
    idx = jax.lax.broadcasted_iota(jnp.int32, dist.shape, 0)
    return jnp.min(jnp.where(dist == m, idx, k), axis=0, keepdims=True)


def _assign_half(x1, x2, c1, c2, c1sq, c2sq):
    dims = (((1,), (1,)), ((), ()))
    mm1 = jax.lax.dot_general(c1, x1, dims,
                              preferred_element_type=jnp.float32)
    dist1 = c1sq - 2.0 * mm1
    outer = _first_argmin_t(dist1, 256)
    mm2 = jax.lax.dot_general(c2, x2, dims,
                              preferred_element_type=jnp.float32)
    dist2 = c2sq - 2.0 * mm2
    inner = _first_argmin_t(dist2, 32)
    return inner + outer * _N_PER


def _cluster_kernel(x1t_ref, x2t_ref, x1b_ref, x2b_ref, c1_ref, c2_ref,
                    out_ref, c1sq_ref, c2sq_ref):
    @pl.when(pl.program_id(0) == 0)
    def _():
        c1 = c1_ref[...]
        c2 = c2_ref[...]
        c1sq_ref[...] = jnp.sum(c1 * c1, axis=1, keepdims=True)
        c2sq_ref[...] = jnp.sum(c2 * c2, axis=1, keepdims=True)

    c1 = c1_ref[...]
    c2 = c2_ref[...]
    c1sq = c1sq_ref[...]
    c2sq = c2sq_ref[...]
    out_ref[0, :, :_HALF] = _assign_half(
        x1t_ref[...], x2t_ref[...], c1, c2, c1sq, c2sq)
    out_ref[0, :, _HALF:] = _assign_half(
        x1b_ref[...], x2b_ref[...], c1, c2, c1sq, c2sq)


@jax.jit
def kernel(x, centers1, centers2):
    n = x.shape[0]
    grid = n // _BLOCK

    def xspec(r, c):
        return pl.BlockSpec((_HALF, 128), lambda i, r=r, c=c: (2 * i + r, c))

    out = pl.pallas_call(
        _cluster_kernel,
        grid=(grid,),
        in_specs=[
            xspec(0, 0), xspec(0, 1), xspec(1, 0), xspec(1, 1),
            pl.BlockSpec((256, 128), lambda i: (0, 0)),
            pl.BlockSpec((_N_PER, 128), lambda i: (0, 0)),
        ],
        out_specs=pl.BlockSpec((1, 1, _BLOCK), lambda i: (i, 0, 0)),
        out_shape=jax.ShapeDtypeStruct((grid, 1, _BLOCK), jnp.int32),
        scratch_shapes=[
            pltpu.VMEM((256, 1), jnp.float32),
            pltpu.VMEM((_N_PER, 1), jnp.float32),
        ],
    )(x, x, x, x, centers1, centers2)
    return out.reshape(n)


# speedup vs baseline: 1.0283x; 1.0283x over previous
import jax
import jax.numpy as jnp
from jax.experimental import pallas as pl
from jax.experimental.pallas import tpu as pltpu

_N_PER = 32
_BLOCK = 8192


def _first_argmin_t(dist, k):
    m = jnp.min(dist, axis=0, keepdims=True)
    idx = jax.lax.broadcasted_iota(jnp.int32, dist.shape, 0)
    return jnp.min(jnp.where(dist == m, idx, k), axis=0, keepdims=True)


def _cluster_kernel(x_ref, c1_ref, c2_ref, out_ref, c1sq_ref, c2sq_ref):
    @pl.when(pl.program_id(0) == 0)
    def _():
        c1 = c1_ref[...]
        c2 = c2_ref[...]
        c1sq_ref[...] = jnp.sum(c1 * c1, axis=1, keepdims=True)
        c2sq_ref[...] = jnp.sum(c2 * c2, axis=1, keepdims=True)

    dims = (((1,), (1,)), ((), ()))
    mm1 = jax.lax.dot_general(c1_ref[...], x_ref[:, :128], dims,
                              preferred_element_type=jnp.float32)
    dist1 = c1sq_ref[...] - 2.0 * mm1
    outer = _first_argmin_t(dist1, 256)

    mm2 = jax.lax.dot_general(c2_ref[...], x_ref[:, 128:], dims,
                              preferred_element_type=jnp.float32)
    dist2 = c2sq_ref[...] - 2.0 * mm2
    inner = _first_argmin_t(dist2, 32)

    out_ref[0] = inner + outer * _N_PER


@jax.jit
def kernel(x, centers1, centers2):
    n = x.shape[0]
    grid = n // _BLOCK
    out = pl.pallas_call(
        _cluster_kernel,
        grid=(grid,),
        in_specs=[
            pl.BlockSpec((_BLOCK, 256), lambda i: (i, 0)),
            pl.BlockSpec((256, 128), lambda i: (0, 0)),
            pl.BlockSpec((_N_PER, 128), lambda i: (0, 0)),
        ],
        out_specs=pl.BlockSpec((1, 1, _BLOCK), lambda i: (i, 0, 0)),
        out_shape=jax.ShapeDtypeStruct((grid, 1, _BLOCK), jnp.int32),
        scratch_shapes=[
            pltpu.VMEM((256, 1), jnp.float32),
            pltpu.VMEM((_N_PER, 1), jnp.float32),
        ],
    )(x, centers1, centers2)
    return out.reshape(n)
